# baseline (speedup 1.0000x reference)
import functools

import jax
import jax.numpy as jnp
from jax import lax
from jax.experimental import pallas as pl
from jax.experimental.pallas import tpu as pltpu
from jax.experimental.pallas import tpu_sc as plsc

NUM_ACT = 2048
SIZE = 51
RANGE_ = 4.0
GRID = 2.0 * RANGE_ / (SIZE - 1)
HALF = SIZE // 2

NC, NS, L = 2, 16, 16
NW = NC * NS
NHALF = 2
COLS_W = NUM_ACT // NHALF
TECS_PER_HALF = NW // NHALF
ROWS_PER_CHUNK = 4
CHUNK = ROWS_PER_CHUNK * COLS_W

P_SZ = NUM_ACT * SIZE
Q_SZ = NUM_ACT * (SIZE - 1)
PW = COLS_W * SIZE
QW = COLS_W * (SIZE - 1)
TAB_W = PW + QW

MAGIC_F = 12582912.0
MAGIC_BITS = 0x4B400000


def _project_body(cs_ref, p_ref, q_ref):
    cs = cs_ref[...]
    slopes = jnp.clip(cs[:, 1:] - cs[:, :-1], 0.0, jnp.float32(GRID))
    k = lax.broadcasted_iota(jnp.int32, (SIZE - 1, SIZE), 0)
    j = lax.broadcasted_iota(jnp.int32, (SIZE - 1, SIZE), 1)
    m = (j > k).astype(jnp.float32)
    cum = lax.dot_general(
        slopes, m, (((1,), (0,)), ((), ())),
        preferred_element_type=jnp.float32,
        precision=lax.Precision.HIGHEST)
    cv = cum - cum[:, HALF:HALF + 1]
    qext = jnp.concatenate(
        [slopes, jnp.zeros((NUM_ACT, 1), jnp.float32)], axis=1)
    kk = lax.broadcasted_iota(jnp.int32, (NUM_ACT, SIZE), 1).astype(jnp.float32)
    p_ref[...] = cv - (kk - jnp.float32(0.5)) * qext
    q_ref[...] = slopes


def _make_sc_kernel(n_rows):
    rows_w = n_rows // TECS_PER_HALF
    ch_per_w = rows_w // ROWS_PER_CHUNK
    mesh = plsc.VectorSubcoreMesh(
        core_axis_name="c", subcore_axis_name="s",
        num_cores=NC, num_subcores=NS)

    @functools.partial(
        pl.kernel,
        out_type=jax.ShapeDtypeStruct((n_rows, NUM_ACT), jnp.float32),
        mesh=mesh,
        compiler_params=pltpu.CompilerParams(needs_layout_passes=False),
        scratch_types=[
            pltpu.VMEM((TAB_W,), jnp.float32),
            pltpu.VMEM((COLS_W,), jnp.float32),
            pltpu.VMEM((COLS_W,), jnp.float32),
            pltpu.VMEM((ROWS_PER_CHUNK, COLS_W), jnp.float32),
            pltpu.VMEM((ROWS_PER_CHUNK, COLS_W), jnp.float32),
            pltpu.VMEM((ROWS_PER_CHUNK, COLS_W), jnp.float32),
            pltpu.VMEM((ROWS_PER_CHUNK, COLS_W), jnp.float32),
            pltpu.SemaphoreType.DMA,
            pltpu.SemaphoreType.DMA,
            pltpu.SemaphoreType.DMA,
            pltpu.SemaphoreType.DMA,
        ],
    )
    def sc_kernel(x_hbm, p_hbm, q_hbm, pm_hbm, inv_hbm, out_hbm,
                  table, pm_v, inv_v, xb0, xb1, ob0, ob1,
                  sin0, sin1, sout0, sout1):
        wid = lax.axis_index("s") * NC + lax.axis_index("c")
        h = wid // TECS_PER_HALF
        t = wid % TECS_PER_HALF
        pltpu.sync_copy(p_hbm.at[pl.ds(h * PW, PW)], table.at[pl.ds(0, PW)])
        pltpu.sync_copy(q_hbm.at[pl.ds(h * QW, QW)], table.at[pl.ds(PW, QW)])
        pltpu.sync_copy(pm_hbm.at[pl.ds(h * COLS_W, COLS_W)], pm_v)
        pltpu.sync_copy(inv_hbm.at[pl.ds(h * COLS_W, COLS_W)], inv_v)
        col0 = h * COLS_W
        row0 = t * rows_w
        xbs = (xb0, xb1)
        obs = (ob0, ob1)
        sins = (sin0, sin1)
        souts = (sout0, sout1)

        def start_in(c, b):
            r = row0 + c * ROWS_PER_CHUNK
            pltpu.async_copy(
                x_hbm.at[pl.ds(r, ROWS_PER_CHUNK), pl.ds(col0, COLS_W)],
                xbs[b], sins[b])

        def start_out(c, b):
            r = row0 + c * ROWS_PER_CHUNK
            pltpu.async_copy(
                obs[b],
                out_hbm.at[pl.ds(r, ROWS_PER_CHUNK), pl.ds(col0, COLS_W)],
                souts[b])

        def wait_in(b):
            pltpu.make_async_copy(
                x_hbm.at[pl.ds(0, ROWS_PER_CHUNK), pl.ds(0, COLS_W)],
                xbs[b], sins[b]).wait()

        def wait_out(b):
            pltpu.make_async_copy(
                obs[b],
                out_hbm.at[pl.ds(0, ROWS_PER_CHUNK), pl.ds(0, COLS_W)],
                souts[b]).wait()

        iota51 = lax.iota(jnp.int32, L) * SIZE
        iota50 = lax.iota(jnp.int32, L) * (SIZE - 1)

        def compute(b):
            xb = xbs[b]
            ob = obs[b]

            @plsc.parallel_loop(0, COLS_W // L, unroll=3)
            def _(i):
                col = i * L
                pm = pm_v[pl.ds(col, L)]
                iv = inv_v[pl.ds(col, L)]
                pbase = iota51 + (col * SIZE - MAGIC_BITS)
                qbase = iota50 + (col * (SIZE - 1) + PW - MAGIC_BITS)
                for r in range(ROWS_PER_CHUNK):
                    xv = xb[r, pl.ds(col, L)]
                    w = xv * pm + jnp.float32(HALF - 0.5)
                    wc = jnp.minimum(jnp.maximum(w, -0.5),
                                     jnp.float32(SIZE - 2))
                    ti = plsc.bitcast(wc + jnp.float32(MAGIC_F), jnp.int32)
                    p = plsc.load_gather(table, [ti + pbase])
                    q = plsc.load_gather(table, [ti + qbase])
                    ob[r, pl.ds(col, L)] = (p + q * w) * iv

        start_in(0, 0)
        start_in(1, 1)

        @pl.loop(0, ch_per_w, step=2)
        def _(c):
            for b in range(2):
                cc = c + b
                wait_in(b)

                @pl.when(cc >= 2)
                def _():
                    wait_out(b)

                compute(b)
                start_out(cc, b)

                @pl.when(cc + 2 < ch_per_w)
                def _():
                    start_in(cc + 2, b)

        wait_out(0)
        wait_out(1)

    return sc_kernel


def kernel(input, coefficients_vect, scaling_coeffs_vect):
    b, c = input.shape
    cs = coefficients_vect.reshape(NUM_ACT, SIZE)
    p, q = pl.pallas_call(
        _project_body,
        out_shape=[
            jax.ShapeDtypeStruct((NUM_ACT, SIZE), jnp.float32),
            jax.ShapeDtypeStruct((NUM_ACT, SIZE - 1), jnp.float32),
        ],
    )(cs)
    s = scaling_coeffs_vect.reshape(NUM_ACT)
    pm = s * jnp.float32(1.0 / GRID)
    inv = 1.0 / s
    return _make_sc_kernel(b)(input, p.reshape(-1), q.reshape(-1), pm, inv)


# speedup vs baseline: 1.0626x; 1.0626x over previous
import functools

import jax
import jax.numpy as jnp
from jax import lax
from jax.experimental import pallas as pl
from jax.experimental.pallas import tpu as pltpu
from jax.experimental.pallas import tpu_sc as plsc

NUM_ACT = 2048
SIZE = 51
RANGE_ = 4.0
GRID = 2.0 * RANGE_ / (SIZE - 1)
HALF = SIZE // 2

NC, NS, L = 2, 16, 16
NW = NC * NS
NHALF = 2
COLS_W = NUM_ACT // NHALF
TECS_PER_HALF = NW // NHALF
ROWS_PER_CHUNK = 4
CHUNK = ROWS_PER_CHUNK * COLS_W

P_SZ = NUM_ACT * SIZE
Q_SZ = NUM_ACT * (SIZE - 1)
PW = COLS_W * SIZE
QW = COLS_W * (SIZE - 1)
TAB_W = PW + QW

MAGIC_F = 12582912.0
MAGIC_BITS = 0x4B400000


def _project_body(cs_ref, p_ref, q_ref):
    cs = cs_ref[...]
    slopes = jnp.clip(cs[:, 1:] - cs[:, :-1], 0.0, jnp.float32(GRID))
    k = lax.broadcasted_iota(jnp.int32, (SIZE - 1, SIZE), 0)
    j = lax.broadcasted_iota(jnp.int32, (SIZE - 1, SIZE), 1)
    m = (j > k).astype(jnp.float32)
    cum = lax.dot_general(
        slopes, m, (((1,), (0,)), ((), ())),
        preferred_element_type=jnp.float32,
        precision=lax.Precision.HIGHEST)
    cv = cum - cum[:, HALF:HALF + 1]
    qext = jnp.concatenate(
        [slopes, jnp.zeros((NUM_ACT, 1), jnp.float32)], axis=1)
    kk = lax.broadcasted_iota(jnp.int32, (NUM_ACT, SIZE), 1).astype(jnp.float32)
    p_ref[...] = cv - (kk - jnp.float32(0.5)) * qext
    q_ref[...] = slopes


def _make_sc_kernel(n_rows):
    rows_w = n_rows // TECS_PER_HALF
    ch_per_w = rows_w // ROWS_PER_CHUNK
    mesh = plsc.VectorSubcoreMesh(
        core_axis_name="c", subcore_axis_name="s",
        num_cores=NC, num_subcores=NS)

    @functools.partial(
        pl.kernel,
        out_type=jax.ShapeDtypeStruct((n_rows, NUM_ACT), jnp.float32),
        mesh=mesh,
        compiler_params=pltpu.CompilerParams(needs_layout_passes=False),
        scratch_types=[
            pltpu.VMEM((TAB_W,), jnp.float32),
            pltpu.VMEM((COLS_W,), jnp.float32),
            pltpu.VMEM((COLS_W,), jnp.float32),
            pltpu.VMEM((ROWS_PER_CHUNK, COLS_W), jnp.float32),
            pltpu.VMEM((ROWS_PER_CHUNK, COLS_W), jnp.float32),
            pltpu.VMEM((ROWS_PER_CHUNK, COLS_W), jnp.float32),
            pltpu.VMEM((ROWS_PER_CHUNK, COLS_W), jnp.float32),
            pltpu.SemaphoreType.DMA,
            pltpu.SemaphoreType.DMA,
            pltpu.SemaphoreType.DMA,
            pltpu.SemaphoreType.DMA,
        ],
    )
    def sc_kernel(x_hbm, p_hbm, q_hbm, pm_hbm, inv_hbm, out_hbm,
                  table, pm_v, inv_v, xb0, xb1, ob0, ob1,
                  sin0, sin1, sout0, sout1):
        wid = lax.axis_index("s") * NC + lax.axis_index("c")
        h = wid // TECS_PER_HALF
        t = wid % TECS_PER_HALF
        pltpu.async_copy(p_hbm.at[pl.ds(h * PW, PW)],
                         table.at[pl.ds(0, PW)], sout0)
        pltpu.async_copy(q_hbm.at[pl.ds(h * QW, QW)],
                         table.at[pl.ds(PW, QW)], sout1)
        pltpu.async_copy(pm_hbm.at[pl.ds(h * COLS_W, COLS_W)], pm_v, sout0)
        pltpu.async_copy(inv_hbm.at[pl.ds(h * COLS_W, COLS_W)], inv_v, sout1)
        col0 = h * COLS_W
        row0 = t * rows_w
        xbs = (xb0, xb1)
        obs = (ob0, ob1)
        sins = (sin0, sin1)
        souts = (sout0, sout1)

        def start_in(c, b):
            r = row0 + c * ROWS_PER_CHUNK
            pltpu.async_copy(
                x_hbm.at[pl.ds(r, ROWS_PER_CHUNK), pl.ds(col0, COLS_W)],
                xbs[b], sins[b])

        def start_out(c, b):
            r = row0 + c * ROWS_PER_CHUNK
            pltpu.async_copy(
                obs[b],
                out_hbm.at[pl.ds(r, ROWS_PER_CHUNK), pl.ds(col0, COLS_W)],
                souts[b])

        def wait_in(b):
            pltpu.make_async_copy(
                x_hbm.at[pl.ds(0, ROWS_PER_CHUNK), pl.ds(0, COLS_W)],
                xbs[b], sins[b]).wait()

        def wait_out(b):
            pltpu.make_async_copy(
                obs[b],
                out_hbm.at[pl.ds(0, ROWS_PER_CHUNK), pl.ds(0, COLS_W)],
                souts[b]).wait()

        iota51 = lax.iota(jnp.int32, L) * SIZE
        iota50 = lax.iota(jnp.int32, L) * (SIZE - 1)

        def compute(b):
            xb = xbs[b]
            ob = obs[b]

            @plsc.parallel_loop(0, COLS_W // L, unroll=2)
            def _(i):
                col = i * L
                pm = pm_v[pl.ds(col, L)]
                iv = inv_v[pl.ds(col, L)]
                pbase = iota51 + (col * SIZE - MAGIC_BITS)
                qbase = iota50 + (col * (SIZE - 1) + PW - MAGIC_BITS)
                for r in range(ROWS_PER_CHUNK):
                    xv = xb[r, pl.ds(col, L)]
                    w = xv * pm + jnp.float32(HALF - 0.5)
                    wc = jnp.minimum(jnp.maximum(w, -0.5),
                                     jnp.float32(SIZE - 2))
                    ti = plsc.bitcast(wc + jnp.float32(MAGIC_F), jnp.int32)
                    p = plsc.load_gather(table, [ti + pbase])
                    q = plsc.load_gather(table, [ti + qbase])
                    ob[r, pl.ds(col, L)] = (p + q * w) * iv

        start_in(0, 0)
        start_in(1, 1)
        pltpu.make_async_copy(p_hbm.at[pl.ds(0, PW)],
                              table.at[pl.ds(0, PW)], sout0).wait()
        pltpu.make_async_copy(q_hbm.at[pl.ds(0, QW)],
                              table.at[pl.ds(PW, QW)], sout1).wait()
        pltpu.make_async_copy(pm_hbm.at[pl.ds(0, COLS_W)], pm_v, sout0).wait()
        pltpu.make_async_copy(inv_hbm.at[pl.ds(0, COLS_W)], inv_v, sout1).wait()

        @pl.loop(0, ch_per_w, step=2)
        def _(c):
            for b in range(2):
                cc = c + b
                wait_in(b)

                @pl.when(cc >= 2)
                def _():
                    wait_out(b)

                compute(b)
                start_out(cc, b)

                @pl.when(cc + 2 < ch_per_w)
                def _():
                    start_in(cc + 2, b)

        wait_out(0)
        wait_out(1)

    return sc_kernel


def kernel(input, coefficients_vect, scaling_coeffs_vect):
    b, c = input.shape
    cs = coefficients_vect.reshape(NUM_ACT, SIZE)
    p, q = pl.pallas_call(
        _project_body,
        out_shape=[
            jax.ShapeDtypeStruct((NUM_ACT, SIZE), jnp.float32),
            jax.ShapeDtypeStruct((NUM_ACT, SIZE - 1), jnp.float32),
        ],
    )(cs)
    s = scaling_coeffs_vect.reshape(NUM_ACT)
    pm = s * jnp.float32(1.0 / GRID)
    inv = 1.0 / s
    return _make_sc_kernel(b)(input, p.reshape(-1), q.reshape(-1), pm, inv)
